# baseline (speedup 1.0000x reference)
import functools

import jax
import jax.numpy as jnp
from jax.experimental import pallas as pl
from jax.experimental.pallas import tpu as pltpu

_F32 = jnp.float32
_VMEM_LIMIT = 60 * 1024 * 1024
_NBANK = 4
_GATHER_UNROLL = 16


def _tile_spec(block_shape):
    nd = len(block_shape)
    return pl.BlockSpec(block_shape, lambda i, _nd=nd: (i,) + (0,) * (_nd - 1))


def _const_spec(shape):
    nd = len(shape)
    return pl.BlockSpec(shape, lambda i, _nd=nd: (0,) * _nd)


def _edge_mlp_body(ea_ref, k1w_ref, k1b_ref, k2w_ref, k2b_ref, o_ref):
    e = jnp.dot(ea_ref[...], k1w_ref[...], preferred_element_type=_F32) + k1b_ref[...]
    e = jnp.maximum(e, 0.0)
    e = jnp.dot(e, k2w_ref[...], preferred_element_type=_F32) + k2b_ref[...]
    o_ref[...] = jnp.maximum(e, 0.0)


def _edge_mlp(ea, k1w, k1b, k2w, k2b, *, tile):
    e_pad, k_pad = ea.shape
    wk = k2w.shape[1]
    return pl.pallas_call(
        _edge_mlp_body,
        out_shape=jax.ShapeDtypeStruct((e_pad, wk), _F32),
        grid=(e_pad // tile,),
        in_specs=[_tile_spec((tile, k_pad)),
                  _const_spec(k1w.shape), _const_spec(k1b.shape),
                  _const_spec(k2w.shape), _const_spec(k2b.shape)],
        out_specs=_tile_spec((tile, wk)),
        compiler_params=pltpu.CompilerParams(
            dimension_semantics=("parallel",),
            vmem_limit_bytes=_VMEM_LIMIT,
        ),
    )(ea, k1w, k1b, k2w, k2b)


def _msg_body(r2_ref, src_ref, tgt_ref, sc_ref, h_ref, k3w_ref, k3b_ref,
              rep_ref, o_ref, xs_scr, msg_scr, acc_scr, dma_sem):
    tile = msg_scr.shape[0]
    p = pl.program_id(0)
    t = pl.program_id(1)
    half = pl.num_programs(1)

    @pl.when(t == 0)
    def _zero():
        acc_scr[...] = jnp.zeros_like(acc_scr)

    def gather_chunk(ci, carry):
        base = ci * _GATHER_UNROLL
        for u in range(_GATHER_UNROLL):
            idx = src_ref[0, 0, base + u]
            xs_scr[pl.ds(base + u, 1), :] = h_ref[pl.ds(idx, 1), :]
        return carry

    jax.lax.fori_loop(0, tile // _GATHER_UNROLL, gather_chunk, 0)

    wflat = jnp.dot(r2_ref[...], k3w_ref[...], preferred_element_type=_F32)
    wflat = wflat + k3b_ref[...]
    xr = jnp.dot(xs_scr[...], rep_ref[...], preferred_element_type=_F32)
    prod = xr * wflat
    s = prod[:, 0:128]
    for k in range(1, 8):
        s = s + prod[:, 128 * k:128 * (k + 1)]
    msg = (s[:, 0:32] + s[:, 32:64]) + (s[:, 64:96] + s[:, 96:128])
    msg_scr[...] = msg * sc_ref[...]

    def scatter_chunk(ci, carry):
        base = ci * _NBANK
        vals = []
        for u in range(_NBANK):
            tg = tgt_ref[0, 0, base + u]
            vals.append(acc_scr[pl.ds(tg, 1), 32 * u:32 * (u + 1)] +
                        msg_scr[pl.ds(base + u, 1), :])
        for u in range(_NBANK):
            tg = tgt_ref[0, 0, base + u]
            acc_scr[pl.ds(tg, 1), 32 * u:32 * (u + 1)] = vals[u]
        return carry

    jax.lax.fori_loop(0, tile // _NBANK, scatter_chunk, 0)

    @pl.when(t == half - 1)
    def _flush():
        copy = pltpu.make_async_copy(acc_scr, o_ref.at[p], dma_sem)
        copy.start()
        copy.wait()


def _messages_aggr(relu2, src3d, tgt3d, scale, h, k3w, k3b, rep, *, tile):
    e_pad = relu2.shape[0]
    n, width = h.shape
    n_tiles = e_pad // tile
    half = n_tiles // 2
    return pl.pallas_call(
        _msg_body,
        out_shape=jax.ShapeDtypeStruct((2, n, _NBANK * width), _F32),
        grid=(2, half),
        in_specs=[
            pl.BlockSpec((tile, relu2.shape[1]),
                         lambda p, t, _h=half: (p * _h + t, 0)),
            pl.BlockSpec((1, 1, tile), lambda p, t, _h=half: (p * _h + t, 0, 0),
                         memory_space=pltpu.SMEM),
            pl.BlockSpec((1, 1, tile), lambda p, t, _h=half: (p * _h + t, 0, 0),
                         memory_space=pltpu.SMEM),
            pl.BlockSpec((tile, 1), lambda p, t, _h=half: (p * _h + t, 0)),
            pl.BlockSpec(h.shape, lambda p, t: (0, 0)),
            pl.BlockSpec(k3w.shape, lambda p, t: (0, 0)),
            pl.BlockSpec(k3b.shape, lambda p, t: (0, 0)),
            pl.BlockSpec(rep.shape, lambda p, t: (0, 0)),
        ],
        out_specs=pl.BlockSpec(memory_space=pl.ANY),
        scratch_shapes=[pltpu.VMEM((tile, width), _F32),
                        pltpu.VMEM((tile, width), _F32),
                        pltpu.VMEM((n, _NBANK * width), _F32),
                        pltpu.SemaphoreType.DMA],
        compiler_params=pltpu.CompilerParams(
            dimension_semantics=("parallel", "arbitrary"),
            vmem_limit_bytes=_VMEM_LIMIT,
        ),
    )(relu2, src3d, tgt3d, scale, h, k3w, k3b, rep)


def _node_body(apply_relu, acc_ref, h_ref, root_ref, bias_ref, o_ref):
    a = acc_ref[0] + acc_ref[1]
    aggr = ((a[:, 0:32] + a[:, 32:64]) + (a[:, 64:96] + a[:, 96:128]))
    h_new = aggr + jnp.dot(h_ref[...], root_ref[...],
                           preferred_element_type=_F32) + bias_ref[...]
    if apply_relu:
        h_new = jnp.maximum(h_new, 0.0)
    o_ref[...] = h_new


def _node_update(acc, h, root, bias2d, *, tile, apply_relu):
    n_pad, width = h.shape
    return pl.pallas_call(
        functools.partial(_node_body, apply_relu),
        out_shape=jax.ShapeDtypeStruct((n_pad, width), _F32),
        grid=(n_pad // tile,),
        in_specs=[pl.BlockSpec((2, tile, _NBANK * width), lambda i: (0, i, 0)),
                  _tile_spec((tile, width)),
                  _const_spec(root.shape),
                  _const_spec(bias2d.shape)],
        out_specs=_tile_spec((tile, width)),
        compiler_params=pltpu.CompilerParams(
            dimension_semantics=("parallel",),
            vmem_limit_bytes=_VMEM_LIMIT,
        ),
    )(acc, h, root, bias2d)


@jax.jit
def _forward(fc1_w, fc1_b, k1_w, k1_b, k2_w, k2_b, k3_w, k3_b, root, bias,
             fc2_w, fc2_b, x, ea, src, tgt, scale):
    depth = 3
    edge_tile = 1024
    node_tile = 2048
    k_pad = ea.shape[1]
    ker_in = k1_w.shape[0]

    k1w = jnp.pad(k1_w, ((0, k_pad - ker_in), (0, 0)))
    k1b = k1_b.reshape(1, -1)
    k2b = k2_b.reshape(1, -1)
    k3b = k3_b.reshape(1, -1)
    bias2d = bias.reshape(1, -1)

    relu2 = _edge_mlp(ea, k1w, k1b, k2_w, k2b, tile=4096)

    width = root.shape[0]
    j = jnp.arange(width * width)
    rep = (jnp.arange(width)[:, None] == (j // width)[None, :]).astype(_F32)

    h = x * fc1_w[0][None, :] + fc1_b[None, :]

    src3d = src.reshape(-1, 1, edge_tile)
    tgt3d = tgt.reshape(-1, 1, edge_tile)

    for d in range(depth):
        acc = _messages_aggr(relu2, src3d, tgt3d, scale, h, k3_w, k3b, rep,
                             tile=edge_tile)
        h = _node_update(acc, h, root, bias2d,
                         tile=node_tile, apply_relu=(d != depth - 1))

    return h @ fc2_w + fc2_b[None, :]


def kernel(fc1_w, fc1_b, k1_w, k1_b, k2_w, k2_b, k3_w, k3_b, root, bias,
           fc2_w, fc2_b, x, ea, src, tgt, scale):
    return _forward(fc1_w, fc1_b, k1_w, k1_b, k2_w, k2_b, k3_w, k3_b, root,
                    bias, fc2_w, fc2_b, x, ea, src, tgt, scale)


# speedup vs baseline: 4.3840x; 4.3840x over previous
import functools

import jax
import jax.numpy as jnp
from jax.experimental import pallas as pl
from jax.experimental.pallas import tpu as pltpu

_F32 = jnp.float32
_VMEM_LIMIT = 60 * 1024 * 1024
_NBANK = 4
_GATHER_UNROLL = 16
_SCATTER_UNROLL = 64


def _tile_spec(block_shape):
    nd = len(block_shape)
    return pl.BlockSpec(block_shape, lambda i, _nd=nd: (i,) + (0,) * (_nd - 1))


def _const_spec(shape):
    nd = len(shape)
    return pl.BlockSpec(shape, lambda i, _nd=nd: (0,) * _nd)


def _edge_mlp_body(ea_ref, k1w_ref, k1b_ref, k2w_ref, k2b_ref, o_ref):
    e = jnp.dot(ea_ref[...], k1w_ref[...], preferred_element_type=_F32) + k1b_ref[...]
    e = jnp.maximum(e, 0.0)
    e = jnp.dot(e, k2w_ref[...], preferred_element_type=_F32) + k2b_ref[...]
    o_ref[...] = jnp.maximum(e, 0.0)


def _edge_mlp(ea, k1w, k1b, k2w, k2b, *, tile):
    e_pad, k_pad = ea.shape
    wk = k2w.shape[1]
    return pl.pallas_call(
        _edge_mlp_body,
        out_shape=jax.ShapeDtypeStruct((e_pad, wk), _F32),
        grid=(e_pad // tile,),
        in_specs=[_tile_spec((tile, k_pad)),
                  _const_spec(k1w.shape), _const_spec(k1b.shape),
                  _const_spec(k2w.shape), _const_spec(k2b.shape)],
        out_specs=_tile_spec((tile, wk)),
        compiler_params=pltpu.CompilerParams(
            dimension_semantics=("parallel",),
            vmem_limit_bytes=_VMEM_LIMIT,
        ),
    )(ea, k1w, k1b, k2w, k2b)


def _msg_body(r2_ref, src_ref, tgt_ref, sc_ref, h_ref, k3w_ref, k3b_ref,
              rep_ref, o_ref, xs_scr, msg_scr, acc_scr, dma_sem):
    tile = msg_scr.shape[0]
    p = pl.program_id(0)
    t = pl.program_id(1)
    half = pl.num_programs(1)

    @pl.when(t == 0)
    def _zero():
        acc_scr[...] = jnp.zeros_like(acc_scr)

    def gather_chunk(ci, carry):
        base = ci * _GATHER_UNROLL
        for u in range(_GATHER_UNROLL):
            idx = src_ref[0, 0, base + u]
            xs_scr[pl.ds(base + u, 1), :] = h_ref[pl.ds(idx, 1), :]
        return carry

    jax.lax.fori_loop(0, tile // _GATHER_UNROLL, gather_chunk, 0,
                      unroll=4)

    wflat = jnp.dot(r2_ref[...], k3w_ref[...], preferred_element_type=_F32)
    wflat = wflat + k3b_ref[...]
    xr = jnp.dot(xs_scr[...], rep_ref[...], preferred_element_type=_F32)
    prod = xr * wflat
    s = prod[:, 0:128]
    for k in range(1, 8):
        s = s + prod[:, 128 * k:128 * (k + 1)]
    msg = (s[:, 0:32] + s[:, 32:64]) + (s[:, 64:96] + s[:, 96:128])

    lane_grp = jax.lax.broadcasted_iota(jnp.int32, (tile, 128), 1) // 32
    row_grp = jax.lax.broadcasted_iota(jnp.int32, (tile, 128), 0) % _NBANK
    msgw = msg * sc_ref[...]
    msg4 = jnp.where(lane_grp == row_grp,
                     jnp.concatenate([msgw] * _NBANK, axis=1), 0.0)
    msg_scr[...] = msg4

    def scatter_chunk(ci, carry):
        base = ci * _SCATTER_UNROLL
        for w in range(_SCATTER_UNROLL // _NBANK):
            wb = base + w * _NBANK
            vals = []
            for u in range(_NBANK):
                tg = tgt_ref[0, 0, wb + u]
                vals.append(acc_scr[pl.ds(tg, 1), 32 * u:32 * (u + 1)] +
                            msg_scr[pl.ds(wb + u, 1), 32 * u:32 * (u + 1)])
            for u in range(_NBANK):
                tg = tgt_ref[0, 0, wb + u]
                acc_scr[pl.ds(tg, 1), 32 * u:32 * (u + 1)] = vals[u]
        return carry

    jax.lax.fori_loop(0, tile // _SCATTER_UNROLL, scatter_chunk, 0)

    @pl.when(t == half - 1)
    def _flush():
        copy = pltpu.make_async_copy(acc_scr, o_ref.at[p], dma_sem)
        copy.start()
        copy.wait()


def _messages_aggr(relu2, src3d, tgt3d, scale, h, k3w, k3b, rep, *, tile):
    e_pad = relu2.shape[0]
    n, width = h.shape
    n_tiles = e_pad // tile
    half = n_tiles // 2
    return pl.pallas_call(
        _msg_body,
        out_shape=jax.ShapeDtypeStruct((2, n, _NBANK * width), _F32),
        grid=(2, half),
        in_specs=[
            pl.BlockSpec((tile, relu2.shape[1]),
                         lambda p, t, _h=half: (p * _h + t, 0)),
            pl.BlockSpec((1, 1, tile), lambda p, t, _h=half: (p * _h + t, 0, 0),
                         memory_space=pltpu.SMEM),
            pl.BlockSpec((1, 1, tile), lambda p, t, _h=half: (p * _h + t, 0, 0),
                         memory_space=pltpu.SMEM),
            pl.BlockSpec((tile, 1), lambda p, t, _h=half: (p * _h + t, 0)),
            pl.BlockSpec(h.shape, lambda p, t: (0, 0)),
            pl.BlockSpec(k3w.shape, lambda p, t: (0, 0)),
            pl.BlockSpec(k3b.shape, lambda p, t: (0, 0)),
            pl.BlockSpec(rep.shape, lambda p, t: (0, 0)),
        ],
        out_specs=pl.BlockSpec(memory_space=pl.ANY),
        scratch_shapes=[pltpu.VMEM((tile, width), _F32),
                        pltpu.VMEM((tile, _NBANK * width), _F32),
                        pltpu.VMEM((n, _NBANK * width), _F32),
                        pltpu.SemaphoreType.DMA],
        compiler_params=pltpu.CompilerParams(
            dimension_semantics=("parallel", "arbitrary"),
            vmem_limit_bytes=_VMEM_LIMIT,
        ),
    )(relu2, src3d, tgt3d, scale, h, k3w, k3b, rep)


def _node_body(apply_relu, acc_ref, h_ref, root_ref, bias_ref, o_ref):
    a = acc_ref[0] + acc_ref[1]
    aggr = ((a[:, 0:32] + a[:, 32:64]) + (a[:, 64:96] + a[:, 96:128]))
    h_new = aggr + jnp.dot(h_ref[...], root_ref[...],
                           preferred_element_type=_F32) + bias_ref[...]
    if apply_relu:
        h_new = jnp.maximum(h_new, 0.0)
    o_ref[...] = h_new


def _node_update(acc, h, root, bias2d, *, tile, apply_relu):
    n_pad, width = h.shape
    return pl.pallas_call(
        functools.partial(_node_body, apply_relu),
        out_shape=jax.ShapeDtypeStruct((n_pad, width), _F32),
        grid=(n_pad // tile,),
        in_specs=[pl.BlockSpec((2, tile, _NBANK * width), lambda i: (0, i, 0)),
                  _tile_spec((tile, width)),
                  _const_spec(root.shape),
                  _const_spec(bias2d.shape)],
        out_specs=_tile_spec((tile, width)),
        compiler_params=pltpu.CompilerParams(
            dimension_semantics=("parallel",),
            vmem_limit_bytes=_VMEM_LIMIT,
        ),
    )(acc, h, root, bias2d)


@jax.jit
def _forward(fc1_w, fc1_b, k1_w, k1_b, k2_w, k2_b, k3_w, k3_b, root, bias,
             fc2_w, fc2_b, x, ea, src, tgt, scale):
    depth = 3
    edge_tile = 1024
    node_tile = 2048
    k_pad = ea.shape[1]
    ker_in = k1_w.shape[0]

    k1w = jnp.pad(k1_w, ((0, k_pad - ker_in), (0, 0)))
    k1b = k1_b.reshape(1, -1)
    k2b = k2_b.reshape(1, -1)
    k3b = k3_b.reshape(1, -1)
    bias2d = bias.reshape(1, -1)

    relu2 = _edge_mlp(ea, k1w, k1b, k2_w, k2b, tile=4096)

    width = root.shape[0]
    j = jnp.arange(width * width)
    rep = (jnp.arange(width)[:, None] == (j // width)[None, :]).astype(_F32)

    h = x * fc1_w[0][None, :] + fc1_b[None, :]

    src3d = src.reshape(-1, 1, edge_tile)
    tgt3d = tgt.reshape(-1, 1, edge_tile)

    for d in range(depth):
        acc = _messages_aggr(relu2, src3d, tgt3d, scale, h, k3_w, k3b, rep,
                             tile=edge_tile)
        h = _node_update(acc, h, root, bias2d,
                         tile=node_tile, apply_relu=(d != depth - 1))

    return h @ fc2_w + fc2_b[None, :]


def kernel(fc1_w, fc1_b, k1_w, k1_b, k2_w, k2_b, k3_w, k3_b, root, bias,
           fc2_w, fc2_b, x, ea, src, tgt, scale):
    return _forward(fc1_w, fc1_b, k1_w, k1_b, k2_w, k2_b, k3_w, k3_b, root,
                    bias, fc2_w, fc2_b, x, ea, src, tgt, scale)


# speedup vs baseline: 4.5111x; 1.0290x over previous
import functools

import jax
import jax.numpy as jnp
from jax.experimental import pallas as pl
from jax.experimental.pallas import tpu as pltpu

_F32 = jnp.float32
_VMEM_LIMIT = 60 * 1024 * 1024
_NBANK = 4
_GATHER_UNROLL = 16
_SCATTER_UNROLL = 64


def _tile_spec(block_shape):
    nd = len(block_shape)
    return pl.BlockSpec(block_shape, lambda i, _nd=nd: (i,) + (0,) * (_nd - 1))


def _const_spec(shape):
    nd = len(shape)
    return pl.BlockSpec(shape, lambda i, _nd=nd: (0,) * _nd)


def _edge_mlp_body(ea_ref, k1w_ref, k1b_ref, k2w_ref, k2b_ref, o_ref):
    e = jnp.dot(ea_ref[...], k1w_ref[...], preferred_element_type=_F32) + k1b_ref[...]
    e = jnp.maximum(e, 0.0)
    e = jnp.dot(e, k2w_ref[...], preferred_element_type=_F32) + k2b_ref[...]
    o_ref[...] = jnp.maximum(e, 0.0)


def _edge_mlp(ea, k1w, k1b, k2w, k2b, *, tile):
    e_pad, k_pad = ea.shape
    wk = k2w.shape[1]
    return pl.pallas_call(
        _edge_mlp_body,
        out_shape=jax.ShapeDtypeStruct((e_pad, wk), _F32),
        grid=(e_pad // tile,),
        in_specs=[_tile_spec((tile, k_pad)),
                  _const_spec(k1w.shape), _const_spec(k1b.shape),
                  _const_spec(k2w.shape), _const_spec(k2b.shape)],
        out_specs=_tile_spec((tile, wk)),
        compiler_params=pltpu.CompilerParams(
            dimension_semantics=("parallel",),
            vmem_limit_bytes=_VMEM_LIMIT,
        ),
    )(ea, k1w, k1b, k2w, k2b)


def _scatter_window(acc_a, acc_b, msg_ref, tgt_ref, tslot, wb):
    accs = (acc_a,) * _NBANK + (acc_b,) * _NBANK
    vals = []
    for u in range(2 * _NBANK):
        g = u % _NBANK
        tg = tgt_ref[0, tslot, wb + u]
        vals.append(accs[u][pl.ds(tg, 1), 32 * g:32 * (g + 1)] +
                    msg_ref[pl.ds(wb + u, 1), 32 * g:32 * (g + 1)])
    for u in range(2 * _NBANK):
        g = u % _NBANK
        tg = tgt_ref[0, tslot, wb + u]
        accs[u][pl.ds(tg, 1), 32 * g:32 * (g + 1)] = vals[u]


def _msg_body(r2_ref, src_ref, tgt_ref, tgtp_ref, sc_ref, h_ref, k3w_ref,
              k3b_ref, rep_ref, o_ref, xs_scr, msg_scr, acc_a, acc_b,
              dma_sem):
    tile = xs_scr.shape[0]
    p = pl.program_id(0)
    t = pl.program_id(1)
    half = pl.num_programs(1)
    parity = jax.lax.rem(t, 2)

    @pl.when(t == 0)
    def _zero():
        acc_a[...] = jnp.zeros_like(acc_a)
        acc_b[...] = jnp.zeros_like(acc_b)
        msg_scr[1] = jnp.zeros_like(msg_scr[1])

    prev = msg_scr.at[1 - parity]

    def mixed_chunk(ci, carry):
        base = ci * _SCATTER_UNROLL
        for w in range(_SCATTER_UNROLL // (2 * _NBANK)):
            wb = base + w * 2 * _NBANK
            for u in range(2 * _NBANK):
                idx = src_ref[0, 0, wb + u]
                xs_scr[pl.ds(wb + u, 1), :] = h_ref[pl.ds(idx, 1), :]
            _scatter_window(acc_a, acc_b, prev, tgtp_ref, 0, wb)
        return carry

    jax.lax.fori_loop(0, tile // _SCATTER_UNROLL, mixed_chunk, 0)

    wflat = jnp.dot(r2_ref[...], k3w_ref[...], preferred_element_type=_F32)
    wflat = wflat + k3b_ref[...]
    xr = jnp.dot(xs_scr[...], rep_ref[...], preferred_element_type=_F32)
    prod = xr * wflat
    s = prod[:, 0:128]
    for k in range(1, 8):
        s = s + prod[:, 128 * k:128 * (k + 1)]
    msg = (s[:, 0:32] + s[:, 32:64]) + (s[:, 64:96] + s[:, 96:128])

    lane_grp = jax.lax.broadcasted_iota(jnp.int32, (tile, 128), 1) // 32
    row_grp = jax.lax.broadcasted_iota(jnp.int32, (tile, 128), 0) % _NBANK
    msgw = msg * sc_ref[...]
    msg4 = jnp.where(lane_grp == row_grp,
                     jnp.concatenate([msgw] * _NBANK, axis=1), 0.0)
    msg_scr[parity] = msg4

    @pl.when(t == half - 1)
    def _tail():
        cur = msg_scr.at[parity]

        def tail_chunk(ci, carry):
            base = ci * _SCATTER_UNROLL
            for w in range(_SCATTER_UNROLL // (2 * _NBANK)):
                _scatter_window(acc_a, acc_b, cur, tgt_ref, 0,
                                base + w * 2 * _NBANK)
            return carry

        jax.lax.fori_loop(0, tile // _SCATTER_UNROLL, tail_chunk, 0)
        copy_a = pltpu.make_async_copy(acc_a, o_ref.at[p, 0], dma_sem)
        copy_a.start()
        copy_a.wait()
        copy_b = pltpu.make_async_copy(acc_b, o_ref.at[p, 1], dma_sem)
        copy_b.start()
        copy_b.wait()


def _messages_aggr(relu2, src3d, tgt3d, scale, h, k3w, k3b, rep, *, tile):
    e_pad = relu2.shape[0]
    n, width = h.shape
    n_tiles = e_pad // tile
    half = n_tiles // 2
    return pl.pallas_call(
        _msg_body,
        out_shape=jax.ShapeDtypeStruct((2, 2, n, _NBANK * width), _F32),
        grid=(2, half),
        in_specs=[
            pl.BlockSpec((tile, relu2.shape[1]),
                         lambda p, t, _h=half: (p * _h + t, 0)),
            pl.BlockSpec((1, 1, tile), lambda p, t, _h=half: (p * _h + t, 0, 0),
                         memory_space=pltpu.SMEM),
            pl.BlockSpec((1, 1, tile), lambda p, t, _h=half: (p * _h + t, 0, 0),
                         memory_space=pltpu.SMEM),
            pl.BlockSpec((1, 1, tile),
                         lambda p, t, _h=half: (p * _h + jnp.maximum(t - 1, 0),
                                                0, 0),
                         memory_space=pltpu.SMEM),
            pl.BlockSpec((tile, 1), lambda p, t, _h=half: (p * _h + t, 0)),
            pl.BlockSpec(h.shape, lambda p, t: (0, 0)),
            pl.BlockSpec(k3w.shape, lambda p, t: (0, 0)),
            pl.BlockSpec(k3b.shape, lambda p, t: (0, 0)),
            pl.BlockSpec(rep.shape, lambda p, t: (0, 0)),
        ],
        out_specs=pl.BlockSpec(memory_space=pl.ANY),
        scratch_shapes=[pltpu.VMEM((tile, width), _F32),
                        pltpu.VMEM((2, tile, _NBANK * width), _F32),
                        pltpu.VMEM((n, _NBANK * width), _F32),
                        pltpu.VMEM((n, _NBANK * width), _F32),
                        pltpu.SemaphoreType.DMA],
        compiler_params=pltpu.CompilerParams(
            dimension_semantics=("parallel", "arbitrary"),
            vmem_limit_bytes=_VMEM_LIMIT,
        ),
    )(relu2, src3d, tgt3d, tgt3d, scale, h, k3w, k3b, rep)


def _node_body(apply_relu, acc_ref, h_ref, root_ref, bias_ref, o_ref):
    a = ((acc_ref[0, 0] + acc_ref[0, 1]) +
         (acc_ref[1, 0] + acc_ref[1, 1]))
    aggr = ((a[:, 0:32] + a[:, 32:64]) + (a[:, 64:96] + a[:, 96:128]))
    h_new = aggr + jnp.dot(h_ref[...], root_ref[...],
                           preferred_element_type=_F32) + bias_ref[...]
    if apply_relu:
        h_new = jnp.maximum(h_new, 0.0)
    o_ref[...] = h_new


def _node_update(acc, h, root, bias2d, *, tile, apply_relu):
    n_pad, width = h.shape
    return pl.pallas_call(
        functools.partial(_node_body, apply_relu),
        out_shape=jax.ShapeDtypeStruct((n_pad, width), _F32),
        grid=(n_pad // tile,),
        in_specs=[pl.BlockSpec((2, 2, tile, _NBANK * width),
                               lambda i: (0, 0, i, 0)),
                  _tile_spec((tile, width)),
                  _const_spec(root.shape),
                  _const_spec(bias2d.shape)],
        out_specs=_tile_spec((tile, width)),
        compiler_params=pltpu.CompilerParams(
            dimension_semantics=("parallel",),
            vmem_limit_bytes=_VMEM_LIMIT,
        ),
    )(acc, h, root, bias2d)


@jax.jit
def _forward(fc1_w, fc1_b, k1_w, k1_b, k2_w, k2_b, k3_w, k3_b, root, bias,
             fc2_w, fc2_b, x, ea, src, tgt, scale):
    depth = 3
    edge_tile = 1024
    node_tile = 2048
    k_pad = ea.shape[1]
    ker_in = k1_w.shape[0]

    k1w = jnp.pad(k1_w, ((0, k_pad - ker_in), (0, 0)))
    k1b = k1_b.reshape(1, -1)
    k2b = k2_b.reshape(1, -1)
    k3b = k3_b.reshape(1, -1)
    bias2d = bias.reshape(1, -1)

    relu2 = _edge_mlp(ea, k1w, k1b, k2_w, k2b, tile=4096)

    width = root.shape[0]
    j = jnp.arange(width * width)
    rep = (jnp.arange(width)[:, None] == (j // width)[None, :]).astype(_F32)

    h = x * fc1_w[0][None, :] + fc1_b[None, :]

    src3d = src.reshape(-1, 1, edge_tile)
    tgt3d = tgt.reshape(-1, 1, edge_tile)

    for d in range(depth):
        acc = _messages_aggr(relu2, src3d, tgt3d, scale, h, k3_w, k3b, rep,
                             tile=edge_tile)
        h = _node_update(acc, h, root, bias2d,
                         tile=node_tile, apply_relu=(d != depth - 1))

    return h @ fc2_w + fc2_b[None, :]


def kernel(fc1_w, fc1_b, k1_w, k1_b, k2_w, k2_b, k3_w, k3_b, root, bias,
           fc2_w, fc2_b, x, ea, src, tgt, scale):
    return _forward(fc1_w, fc1_b, k1_w, k1_b, k2_w, k2_b, k3_w, k3_b, root,
                    bias, fc2_w, fc2_b, x, ea, src, tgt, scale)


# speedup vs baseline: 4.5651x; 1.0120x over previous
import functools

import jax
import jax.numpy as jnp
from jax.experimental import pallas as pl
from jax.experimental.pallas import tpu as pltpu

_F32 = jnp.float32
_VMEM_LIMIT = 60 * 1024 * 1024
_NBANK = 4
_GATHER_UNROLL = 16
_SCATTER_UNROLL = 128


def _tile_spec(block_shape):
    nd = len(block_shape)
    return pl.BlockSpec(block_shape, lambda i, _nd=nd: (i,) + (0,) * (_nd - 1))


def _const_spec(shape):
    nd = len(shape)
    return pl.BlockSpec(shape, lambda i, _nd=nd: (0,) * _nd)


def _edge_mlp_body(ea_ref, k1w_ref, k1b_ref, k2w_ref, k2b_ref, o_ref):
    e = jnp.dot(ea_ref[...], k1w_ref[...], preferred_element_type=_F32) + k1b_ref[...]
    e = jnp.maximum(e, 0.0)
    e = jnp.dot(e, k2w_ref[...], preferred_element_type=_F32) + k2b_ref[...]
    o_ref[...] = jnp.maximum(e, 0.0)


def _edge_mlp(ea, k1w, k1b, k2w, k2b, *, tile):
    e_pad, k_pad = ea.shape
    wk = k2w.shape[1]
    return pl.pallas_call(
        _edge_mlp_body,
        out_shape=jax.ShapeDtypeStruct((e_pad, wk), _F32),
        grid=(e_pad // tile,),
        in_specs=[_tile_spec((tile, k_pad)),
                  _const_spec(k1w.shape), _const_spec(k1b.shape),
                  _const_spec(k2w.shape), _const_spec(k2b.shape)],
        out_specs=_tile_spec((tile, wk)),
        compiler_params=pltpu.CompilerParams(
            dimension_semantics=("parallel",),
            vmem_limit_bytes=_VMEM_LIMIT,
        ),
    )(ea, k1w, k1b, k2w, k2b)


def _scatter_window(acc_a, acc_b, msg_ref, tgt_ref, tslot, wb):
    accs = (acc_a,) * _NBANK + (acc_b,) * _NBANK
    vals = []
    for u in range(2 * _NBANK):
        g = u % _NBANK
        tg = tgt_ref[0, tslot, wb + u]
        vals.append(accs[u][pl.ds(tg, 1), 32 * g:32 * (g + 1)] +
                    msg_ref[pl.ds(wb + u, 1), 32 * g:32 * (g + 1)])
    for u in range(2 * _NBANK):
        g = u % _NBANK
        tg = tgt_ref[0, tslot, wb + u]
        accs[u][pl.ds(tg, 1), 32 * g:32 * (g + 1)] = vals[u]


def _msg_body(r2_ref, src_ref, tgt_ref, tgtp_ref, sc_ref, h_ref, k3w_ref,
              k3b_ref, rep_ref, o_ref, xs_scr, msg_scr, acc_a, acc_b,
              dma_sem):
    tile = xs_scr.shape[0]
    p = pl.program_id(0)
    t = pl.program_id(1)
    half = pl.num_programs(1)
    parity = jax.lax.rem(t, 2)

    @pl.when(t == 0)
    def _zero():
        acc_a[...] = jnp.zeros_like(acc_a)
        acc_b[...] = jnp.zeros_like(acc_b)
        msg_scr[1] = jnp.zeros_like(msg_scr[1])

    prev = msg_scr.at[1 - parity]

    def mixed_chunk(ci, carry):
        base = ci * _SCATTER_UNROLL
        for w in range(_SCATTER_UNROLL // (2 * _NBANK)):
            wb = base + w * 2 * _NBANK
            for u in range(2 * _NBANK):
                idx = src_ref[0, 0, wb + u]
                xs_scr[pl.ds(wb + u, 1), :] = h_ref[pl.ds(idx, 1), :]
            _scatter_window(acc_a, acc_b, prev, tgtp_ref, 0, wb)
        return carry

    jax.lax.fori_loop(0, tile // _SCATTER_UNROLL, mixed_chunk, 0)

    wflat = jnp.dot(r2_ref[...], k3w_ref[...], preferred_element_type=_F32)
    wflat = wflat + k3b_ref[...]
    xr = jnp.dot(xs_scr[...], rep_ref[...], preferred_element_type=_F32)
    prod = xr * wflat
    s = prod[:, 0:128]
    for k in range(1, 8):
        s = s + prod[:, 128 * k:128 * (k + 1)]
    msg = (s[:, 0:32] + s[:, 32:64]) + (s[:, 64:96] + s[:, 96:128])

    lane_grp = jax.lax.broadcasted_iota(jnp.int32, (tile, 128), 1) // 32
    row_grp = jax.lax.broadcasted_iota(jnp.int32, (tile, 128), 0) % _NBANK
    msgw = msg * sc_ref[...]
    msg4 = jnp.where(lane_grp == row_grp,
                     jnp.concatenate([msgw] * _NBANK, axis=1), 0.0)
    msg_scr[parity] = msg4

    @pl.when(t == half - 1)
    def _tail():
        cur = msg_scr.at[parity]

        def tail_chunk(ci, carry):
            base = ci * _SCATTER_UNROLL
            for w in range(_SCATTER_UNROLL // (2 * _NBANK)):
                _scatter_window(acc_a, acc_b, cur, tgt_ref, 0,
                                base + w * 2 * _NBANK)
            return carry

        jax.lax.fori_loop(0, tile // _SCATTER_UNROLL, tail_chunk, 0)
        copy_a = pltpu.make_async_copy(acc_a, o_ref.at[p, 0], dma_sem)
        copy_a.start()
        copy_a.wait()
        copy_b = pltpu.make_async_copy(acc_b, o_ref.at[p, 1], dma_sem)
        copy_b.start()
        copy_b.wait()


def _messages_aggr(relu2, src3d, tgt3d, scale, h, k3w, k3b, rep, *, tile):
    e_pad = relu2.shape[0]
    n, width = h.shape
    n_tiles = e_pad // tile
    half = n_tiles // 2
    return pl.pallas_call(
        _msg_body,
        out_shape=jax.ShapeDtypeStruct((2, 2, n, _NBANK * width), _F32),
        grid=(2, half),
        in_specs=[
            pl.BlockSpec((tile, relu2.shape[1]),
                         lambda p, t, _h=half: (p * _h + t, 0)),
            pl.BlockSpec((1, 1, tile), lambda p, t, _h=half: (p * _h + t, 0, 0),
                         memory_space=pltpu.SMEM),
            pl.BlockSpec((1, 1, tile), lambda p, t, _h=half: (p * _h + t, 0, 0),
                         memory_space=pltpu.SMEM),
            pl.BlockSpec((1, 1, tile),
                         lambda p, t, _h=half: (p * _h + jnp.maximum(t - 1, 0),
                                                0, 0),
                         memory_space=pltpu.SMEM),
            pl.BlockSpec((tile, 1), lambda p, t, _h=half: (p * _h + t, 0)),
            pl.BlockSpec(h.shape, lambda p, t: (0, 0)),
            pl.BlockSpec(k3w.shape, lambda p, t: (0, 0)),
            pl.BlockSpec(k3b.shape, lambda p, t: (0, 0)),
            pl.BlockSpec(rep.shape, lambda p, t: (0, 0)),
        ],
        out_specs=pl.BlockSpec(memory_space=pl.ANY),
        scratch_shapes=[pltpu.VMEM((tile, width), _F32),
                        pltpu.VMEM((2, tile, _NBANK * width), _F32),
                        pltpu.VMEM((n, _NBANK * width), _F32),
                        pltpu.VMEM((n, _NBANK * width), _F32),
                        pltpu.SemaphoreType.DMA],
        compiler_params=pltpu.CompilerParams(
            dimension_semantics=("parallel", "arbitrary"),
            vmem_limit_bytes=_VMEM_LIMIT,
        ),
    )(relu2, src3d, tgt3d, tgt3d, scale, h, k3w, k3b, rep)


def _node_body(apply_relu, acc_ref, h_ref, root_ref, bias_ref, o_ref):
    a = ((acc_ref[0, 0] + acc_ref[0, 1]) +
         (acc_ref[1, 0] + acc_ref[1, 1]))
    aggr = ((a[:, 0:32] + a[:, 32:64]) + (a[:, 64:96] + a[:, 96:128]))
    h_new = aggr + jnp.dot(h_ref[...], root_ref[...],
                           preferred_element_type=_F32) + bias_ref[...]
    if apply_relu:
        h_new = jnp.maximum(h_new, 0.0)
    o_ref[...] = h_new


def _node_update(acc, h, root, bias2d, *, tile, apply_relu):
    n_pad, width = h.shape
    return pl.pallas_call(
        functools.partial(_node_body, apply_relu),
        out_shape=jax.ShapeDtypeStruct((n_pad, width), _F32),
        grid=(n_pad // tile,),
        in_specs=[pl.BlockSpec((2, 2, tile, _NBANK * width),
                               lambda i: (0, 0, i, 0)),
                  _tile_spec((tile, width)),
                  _const_spec(root.shape),
                  _const_spec(bias2d.shape)],
        out_specs=_tile_spec((tile, width)),
        compiler_params=pltpu.CompilerParams(
            dimension_semantics=("parallel",),
            vmem_limit_bytes=_VMEM_LIMIT,
        ),
    )(acc, h, root, bias2d)


@jax.jit
def _forward(fc1_w, fc1_b, k1_w, k1_b, k2_w, k2_b, k3_w, k3_b, root, bias,
             fc2_w, fc2_b, x, ea, src, tgt, scale):
    depth = 3
    edge_tile = 1024
    node_tile = 2048
    k_pad = ea.shape[1]
    ker_in = k1_w.shape[0]

    k1w = jnp.pad(k1_w, ((0, k_pad - ker_in), (0, 0)))
    k1b = k1_b.reshape(1, -1)
    k2b = k2_b.reshape(1, -1)
    k3b = k3_b.reshape(1, -1)
    bias2d = bias.reshape(1, -1)

    relu2 = _edge_mlp(ea, k1w, k1b, k2_w, k2b, tile=4096)

    width = root.shape[0]
    j = jnp.arange(width * width)
    rep = (jnp.arange(width)[:, None] == (j // width)[None, :]).astype(_F32)

    h = x * fc1_w[0][None, :] + fc1_b[None, :]

    src3d = src.reshape(-1, 1, edge_tile)
    tgt3d = tgt.reshape(-1, 1, edge_tile)

    for d in range(depth):
        acc = _messages_aggr(relu2, src3d, tgt3d, scale, h, k3_w, k3b, rep,
                             tile=edge_tile)
        h = _node_update(acc, h, root, bias2d,
                         tile=node_tile, apply_relu=(d != depth - 1))

    return h @ fc2_w + fc2_b[None, :]


def kernel(fc1_w, fc1_b, k1_w, k1_b, k2_w, k2_b, k3_w, k3_b, root, bias,
           fc2_w, fc2_b, x, ea, src, tgt, scale):
    return _forward(fc1_w, fc1_b, k1_w, k1_b, k2_w, k2_b, k3_w, k3_b, root,
                    bias, fc2_w, fc2_b, x, ea, src, tgt, scale)


# speedup vs baseline: 4.9187x; 1.0775x over previous
import functools

import jax
import jax.numpy as jnp
from jax.experimental import pallas as pl
from jax.experimental.pallas import tpu as pltpu

_F32 = jnp.float32
_VMEM_LIMIT = 63 * 1024 * 1024
_NBANK = 4
_GATHER_UNROLL = 16
_SCATTER_UNROLL = 128
_CBLK = 1024


def _tile_spec(block_shape):
    nd = len(block_shape)
    return pl.BlockSpec(block_shape, lambda i, _nd=nd: (i,) + (0,) * (_nd - 1))


def _const_spec(shape):
    nd = len(shape)
    return pl.BlockSpec(shape, lambda i, _nd=nd: (0,) * _nd)


def _edge_mlp_body(ea_ref, k1w_ref, k1b_ref, k2w_ref, k2b_ref, o_ref):
    e = jnp.dot(ea_ref[...], k1w_ref[...], preferred_element_type=_F32) + k1b_ref[...]
    e = jnp.maximum(e, 0.0)
    e = jnp.dot(e, k2w_ref[...], preferred_element_type=_F32) + k2b_ref[...]
    o_ref[...] = jnp.maximum(e, 0.0)


def _edge_mlp(ea, k1w, k1b, k2w, k2b, *, tile):
    e_pad, k_pad = ea.shape
    wk = k2w.shape[1]
    return pl.pallas_call(
        _edge_mlp_body,
        out_shape=jax.ShapeDtypeStruct((e_pad, wk), _F32),
        grid=(e_pad // tile,),
        in_specs=[_tile_spec((tile, k_pad)),
                  _const_spec(k1w.shape), _const_spec(k1b.shape),
                  _const_spec(k2w.shape), _const_spec(k2b.shape)],
        out_specs=_tile_spec((tile, wk)),
        compiler_params=pltpu.CompilerParams(
            dimension_semantics=("parallel",),
            vmem_limit_bytes=_VMEM_LIMIT,
        ),
    )(ea, k1w, k1b, k2w, k2b)


def _scatter_window(acc_a, acc_b, msg_ref, tgt_ref, tslot, wb):
    accs = (acc_a,) * _NBANK + (acc_b,) * _NBANK
    vals = []
    for u in range(2 * _NBANK):
        g = u % _NBANK
        tg = tgt_ref[0, tslot, wb + u]
        vals.append(accs[u][pl.ds(tg, 1), 32 * g:32 * (g + 1)] +
                    msg_ref[pl.ds(wb + u, 1), 32 * g:32 * (g + 1)])
    for u in range(2 * _NBANK):
        g = u % _NBANK
        tg = tgt_ref[0, tslot, wb + u]
        accs[u][pl.ds(tg, 1), 32 * g:32 * (g + 1)] = vals[u]


def _msg_body(r2_ref, src_ref, tgt_ref, tgtp_ref, sc_ref, h_ref, k3w_ref,
              k3b_ref, rep_ref, o_ref, xs_scr, msg_scr, acc_a, acc_b,
              dma_sem):
    tile = xs_scr.shape[0]
    p = pl.program_id(0)
    t = pl.program_id(1)
    half = pl.num_programs(1)
    parity = jax.lax.rem(t, 2)

    @pl.when(t == 0)
    def _zero():
        acc_a[...] = jnp.zeros_like(acc_a)
        acc_b[...] = jnp.zeros_like(acc_b)
        msg_scr[1] = jnp.zeros_like(msg_scr[1])

    prev = msg_scr.at[1 - parity]

    def mixed_chunk(ci, carry):
        base = ci * _SCATTER_UNROLL
        for w in range(_SCATTER_UNROLL // (2 * _NBANK)):
            wb = base + w * 2 * _NBANK
            for u in range(2 * _NBANK):
                idx = src_ref[0, 0, wb + u]
                xs_scr[pl.ds(wb + u, 1), :] = h_ref[pl.ds(idx, 1), :]
            _scatter_window(acc_a, acc_b, prev, tgtp_ref, 0, wb)
        return carry

    jax.lax.fori_loop(0, tile // _SCATTER_UNROLL, mixed_chunk, 0)

    cb_n = tile // _CBLK
    lane_grp = jax.lax.broadcasted_iota(jnp.int32, (_CBLK, 128), 1) // 32
    row_grp = jax.lax.broadcasted_iota(jnp.int32, (_CBLK, 128), 0) % _NBANK
    grp_mask = lane_grp == row_grp
    for cb in range(cb_n):
        sl = slice(cb * _CBLK, (cb + 1) * _CBLK)
        wflat = jnp.dot(r2_ref[sl, :], k3w_ref[...],
                        preferred_element_type=_F32) + k3b_ref[...]
        xr = jnp.dot(xs_scr[sl, :], rep_ref[...], preferred_element_type=_F32)
        prod = xr * wflat
        s = prod[:, 0:128]
        for k in range(1, 8):
            s = s + prod[:, 128 * k:128 * (k + 1)]
        msg = (s[:, 0:32] + s[:, 32:64]) + (s[:, 64:96] + s[:, 96:128])
        msgw = msg * sc_ref[sl, :]
        msg4 = jnp.where(grp_mask,
                         jnp.concatenate([msgw] * _NBANK, axis=1), 0.0)
        msg_scr[parity, sl, :] = msg4

    @pl.when(t == half - 1)
    def _tail():
        cur = msg_scr.at[parity]

        def tail_chunk(ci, carry):
            base = ci * _SCATTER_UNROLL
            for w in range(_SCATTER_UNROLL // (2 * _NBANK)):
                _scatter_window(acc_a, acc_b, cur, tgt_ref, 0,
                                base + w * 2 * _NBANK)
            return carry

        jax.lax.fori_loop(0, tile // _SCATTER_UNROLL, tail_chunk, 0)
        copy_a = pltpu.make_async_copy(acc_a, o_ref.at[p, 0], dma_sem)
        copy_a.start()
        copy_a.wait()
        copy_b = pltpu.make_async_copy(acc_b, o_ref.at[p, 1], dma_sem)
        copy_b.start()
        copy_b.wait()


def _messages_aggr(relu2, src3d, tgt3d, scale, h, k3w, k3b, rep, *, tile):
    e_pad = relu2.shape[0]
    n, width = h.shape
    n_tiles = e_pad // tile
    half = n_tiles // 2
    return pl.pallas_call(
        _msg_body,
        out_shape=jax.ShapeDtypeStruct((2, 2, n, _NBANK * width), _F32),
        grid=(2, half),
        in_specs=[
            pl.BlockSpec((tile, relu2.shape[1]),
                         lambda p, t, _h=half: (p * _h + t, 0)),
            pl.BlockSpec((1, 1, tile), lambda p, t, _h=half: (p * _h + t, 0, 0),
                         memory_space=pltpu.SMEM),
            pl.BlockSpec((1, 1, tile), lambda p, t, _h=half: (p * _h + t, 0, 0),
                         memory_space=pltpu.SMEM),
            pl.BlockSpec((1, 1, tile),
                         lambda p, t, _h=half: (p * _h + jnp.maximum(t - 1, 0),
                                                0, 0),
                         memory_space=pltpu.SMEM),
            pl.BlockSpec((tile, 1), lambda p, t, _h=half: (p * _h + t, 0)),
            pl.BlockSpec(h.shape, lambda p, t: (0, 0)),
            pl.BlockSpec(k3w.shape, lambda p, t: (0, 0)),
            pl.BlockSpec(k3b.shape, lambda p, t: (0, 0)),
            pl.BlockSpec(rep.shape, lambda p, t: (0, 0)),
        ],
        out_specs=pl.BlockSpec(memory_space=pl.ANY),
        scratch_shapes=[pltpu.VMEM((tile, width), _F32),
                        pltpu.VMEM((2, tile, _NBANK * width), _F32),
                        pltpu.VMEM((n, _NBANK * width), _F32),
                        pltpu.VMEM((n, _NBANK * width), _F32),
                        pltpu.SemaphoreType.DMA],
        compiler_params=pltpu.CompilerParams(
            dimension_semantics=("parallel", "arbitrary"),
            vmem_limit_bytes=_VMEM_LIMIT,
        ),
    )(relu2, src3d, tgt3d, tgt3d, scale, h, k3w, k3b, rep)


def _node_body(apply_relu, acc_ref, h_ref, root_ref, bias_ref, o_ref):
    a = ((acc_ref[0, 0] + acc_ref[0, 1]) +
         (acc_ref[1, 0] + acc_ref[1, 1]))
    aggr = ((a[:, 0:32] + a[:, 32:64]) + (a[:, 64:96] + a[:, 96:128]))
    h_new = aggr + jnp.dot(h_ref[...], root_ref[...],
                           preferred_element_type=_F32) + bias_ref[...]
    if apply_relu:
        h_new = jnp.maximum(h_new, 0.0)
    o_ref[...] = h_new


def _node_update(acc, h, root, bias2d, *, tile, apply_relu):
    n_pad, width = h.shape
    return pl.pallas_call(
        functools.partial(_node_body, apply_relu),
        out_shape=jax.ShapeDtypeStruct((n_pad, width), _F32),
        grid=(n_pad // tile,),
        in_specs=[pl.BlockSpec((2, 2, tile, _NBANK * width),
                               lambda i: (0, 0, i, 0)),
                  _tile_spec((tile, width)),
                  _const_spec(root.shape),
                  _const_spec(bias2d.shape)],
        out_specs=_tile_spec((tile, width)),
        compiler_params=pltpu.CompilerParams(
            dimension_semantics=("parallel",),
            vmem_limit_bytes=_VMEM_LIMIT,
        ),
    )(acc, h, root, bias2d)


@jax.jit
def _forward(fc1_w, fc1_b, k1_w, k1_b, k2_w, k2_b, k3_w, k3_b, root, bias,
             fc2_w, fc2_b, x, ea, src, tgt, scale):
    depth = 3
    edge_tile = 2048
    node_tile = 2048
    k_pad = ea.shape[1]
    ker_in = k1_w.shape[0]

    k1w = jnp.pad(k1_w, ((0, k_pad - ker_in), (0, 0)))
    k1b = k1_b.reshape(1, -1)
    k2b = k2_b.reshape(1, -1)
    k3b = k3_b.reshape(1, -1)
    bias2d = bias.reshape(1, -1)

    relu2 = _edge_mlp(ea, k1w, k1b, k2_w, k2b, tile=4096)

    width = root.shape[0]
    j = jnp.arange(width * width)
    rep = (jnp.arange(width)[:, None] == (j // width)[None, :]).astype(_F32)

    h = x * fc1_w[0][None, :] + fc1_b[None, :]

    src3d = src.reshape(-1, 1, edge_tile)
    tgt3d = tgt.reshape(-1, 1, edge_tile)

    for d in range(depth):
        acc = _messages_aggr(relu2, src3d, tgt3d, scale, h, k3_w, k3b, rep,
                             tile=edge_tile)
        h = _node_update(acc, h, root, bias2d,
                         tile=node_tile, apply_relu=(d != depth - 1))

    return h @ fc2_w + fc2_b[None, :]


def kernel(fc1_w, fc1_b, k1_w, k1_b, k2_w, k2_b, k3_w, k3_b, root, bias,
           fc2_w, fc2_b, x, ea, src, tgt, scale):
    return _forward(fc1_w, fc1_b, k1_w, k1_b, k2_w, k2_b, k3_w, k3_b, root,
                    bias, fc2_w, fc2_b, x, ea, src, tgt, scale)


# speedup vs baseline: 4.9250x; 1.0013x over previous
import functools

import jax
import jax.numpy as jnp
from jax.experimental import pallas as pl
from jax.experimental.pallas import tpu as pltpu

_F32 = jnp.float32
_VMEM_LIMIT = 63 * 1024 * 1024
_NBANK = 4
_GATHER_UNROLL = 16
_SCATTER_UNROLL = 128
_CBLK = 1024


def _tile_spec(block_shape):
    nd = len(block_shape)
    return pl.BlockSpec(block_shape, lambda i, _nd=nd: (i,) + (0,) * (_nd - 1))


def _const_spec(shape):
    nd = len(shape)
    return pl.BlockSpec(shape, lambda i, _nd=nd: (0,) * _nd)


def _edge_mlp_body(ea_ref, k1w_ref, k1b_ref, k2w_ref, k2b_ref, o_ref):
    e = jnp.dot(ea_ref[...], k1w_ref[...], preferred_element_type=_F32) + k1b_ref[...]
    e = jnp.maximum(e, 0.0)
    e = jnp.dot(e, k2w_ref[...], preferred_element_type=_F32) + k2b_ref[...]
    o_ref[...] = jnp.maximum(e, 0.0)


def _edge_mlp(ea, k1w, k1b, k2w, k2b, *, tile):
    e_pad, k_pad = ea.shape
    wk = k2w.shape[1]
    return pl.pallas_call(
        _edge_mlp_body,
        out_shape=jax.ShapeDtypeStruct((e_pad, wk), _F32),
        grid=(e_pad // tile,),
        in_specs=[_tile_spec((tile, k_pad)),
                  _const_spec(k1w.shape), _const_spec(k1b.shape),
                  _const_spec(k2w.shape), _const_spec(k2b.shape)],
        out_specs=_tile_spec((tile, wk)),
        compiler_params=pltpu.CompilerParams(
            dimension_semantics=("parallel",),
            vmem_limit_bytes=_VMEM_LIMIT,
        ),
    )(ea, k1w, k1b, k2w, k2b)


def _scatter_window(acc_a, acc_b, msg_ref, tgt_ref, tslot, wb):
    accs = (acc_a,) * _NBANK + (acc_b,) * _NBANK
    vals = []
    for u in range(2 * _NBANK):
        g = u % _NBANK
        tg = tgt_ref[0, tslot, wb + u]
        vals.append(accs[u][pl.ds(tg, 1), 32 * g:32 * (g + 1)] +
                    msg_ref[pl.ds(wb + u, 1), 32 * g:32 * (g + 1)])
    for u in range(2 * _NBANK):
        g = u % _NBANK
        tg = tgt_ref[0, tslot, wb + u]
        accs[u][pl.ds(tg, 1), 32 * g:32 * (g + 1)] = vals[u]


def _msg_body(r2_ref, src_ref, tgt_ref, tgtp_ref, sc_ref, h_ref, k3w_ref,
              k3b_ref, rep_ref, o_ref, xs_scr, msg_scr, acc_a, acc_b,
              dma_sem):
    tile = xs_scr.shape[0]
    p = pl.program_id(0)
    t = pl.program_id(1)
    half = pl.num_programs(1)
    parity = jax.lax.rem(t, 2)

    @pl.when(t == 0)
    def _zero():
        acc_a[...] = jnp.zeros_like(acc_a)
        acc_b[...] = jnp.zeros_like(acc_b)
        msg_scr[1] = jnp.zeros_like(msg_scr[1])

    prev = msg_scr.at[1 - parity]

    cb_n = tile // _CBLK
    lane_grp = jax.lax.broadcasted_iota(jnp.int32, (_CBLK, 128), 1) // 32
    row_grp = jax.lax.broadcasted_iota(jnp.int32, (_CBLK, 128), 0) % _NBANK
    grp_mask = lane_grp == row_grp
    for cb in range(cb_n):
        sl = slice(cb * _CBLK, (cb + 1) * _CBLK)
        for w in range(_CBLK // (2 * _NBANK)):
            wb = cb * _CBLK + w * 2 * _NBANK
            for u in range(2 * _NBANK):
                idx = src_ref[0, 0, wb + u]
                xs_scr[pl.ds(wb + u, 1), :] = h_ref[pl.ds(idx, 1), :]
            _scatter_window(acc_a, acc_b, prev, tgtp_ref, 0, wb)
        wflat = jnp.dot(r2_ref[sl, :], k3w_ref[...],
                        preferred_element_type=_F32) + k3b_ref[...]
        xr = jnp.dot(xs_scr[sl, :], rep_ref[...], preferred_element_type=_F32)
        prod = xr * wflat
        s = prod[:, 0:128]
        for k in range(1, 8):
            s = s + prod[:, 128 * k:128 * (k + 1)]
        msg = (s[:, 0:32] + s[:, 32:64]) + (s[:, 64:96] + s[:, 96:128])
        msgw = msg * sc_ref[sl, :]
        msg4 = jnp.where(grp_mask,
                         jnp.concatenate([msgw] * _NBANK, axis=1), 0.0)
        msg_scr[parity, sl, :] = msg4

    @pl.when(t == half - 1)
    def _tail():
        cur = msg_scr.at[parity]

        def tail_chunk(ci, carry):
            base = ci * _SCATTER_UNROLL
            for w in range(_SCATTER_UNROLL // (2 * _NBANK)):
                _scatter_window(acc_a, acc_b, cur, tgt_ref, 0,
                                base + w * 2 * _NBANK)
            return carry

        jax.lax.fori_loop(0, tile // _SCATTER_UNROLL, tail_chunk, 0)
        copy_a = pltpu.make_async_copy(acc_a, o_ref.at[p, 0], dma_sem)
        copy_a.start()
        copy_a.wait()
        copy_b = pltpu.make_async_copy(acc_b, o_ref.at[p, 1], dma_sem)
        copy_b.start()
        copy_b.wait()


def _messages_aggr(relu2, src3d, tgt3d, scale, h, k3w, k3b, rep, *, tile):
    e_pad = relu2.shape[0]
    n, width = h.shape
    n_tiles = e_pad // tile
    half = n_tiles // 2
    return pl.pallas_call(
        _msg_body,
        out_shape=jax.ShapeDtypeStruct((2, 2, n, _NBANK * width), _F32),
        grid=(2, half),
        in_specs=[
            pl.BlockSpec((tile, relu2.shape[1]),
                         lambda p, t, _h=half: (p * _h + t, 0)),
            pl.BlockSpec((1, 1, tile), lambda p, t, _h=half: (p * _h + t, 0, 0),
                         memory_space=pltpu.SMEM),
            pl.BlockSpec((1, 1, tile), lambda p, t, _h=half: (p * _h + t, 0, 0),
                         memory_space=pltpu.SMEM),
            pl.BlockSpec((1, 1, tile),
                         lambda p, t, _h=half: (p * _h + jnp.maximum(t - 1, 0),
                                                0, 0),
                         memory_space=pltpu.SMEM),
            pl.BlockSpec((tile, 1), lambda p, t, _h=half: (p * _h + t, 0)),
            pl.BlockSpec(h.shape, lambda p, t: (0, 0)),
            pl.BlockSpec(k3w.shape, lambda p, t: (0, 0)),
            pl.BlockSpec(k3b.shape, lambda p, t: (0, 0)),
            pl.BlockSpec(rep.shape, lambda p, t: (0, 0)),
        ],
        out_specs=pl.BlockSpec(memory_space=pl.ANY),
        scratch_shapes=[pltpu.VMEM((tile, width), _F32),
                        pltpu.VMEM((2, tile, _NBANK * width), _F32),
                        pltpu.VMEM((n, _NBANK * width), _F32),
                        pltpu.VMEM((n, _NBANK * width), _F32),
                        pltpu.SemaphoreType.DMA],
        compiler_params=pltpu.CompilerParams(
            dimension_semantics=("parallel", "arbitrary"),
            vmem_limit_bytes=_VMEM_LIMIT,
        ),
    )(relu2, src3d, tgt3d, tgt3d, scale, h, k3w, k3b, rep)


def _node_body(apply_relu, acc_ref, h_ref, root_ref, bias_ref, o_ref):
    a = ((acc_ref[0, 0] + acc_ref[0, 1]) +
         (acc_ref[1, 0] + acc_ref[1, 1]))
    aggr = ((a[:, 0:32] + a[:, 32:64]) + (a[:, 64:96] + a[:, 96:128]))
    h_new = aggr + jnp.dot(h_ref[...], root_ref[...],
                           preferred_element_type=_F32) + bias_ref[...]
    if apply_relu:
        h_new = jnp.maximum(h_new, 0.0)
    o_ref[...] = h_new


def _node_update(acc, h, root, bias2d, *, tile, apply_relu):
    n_pad, width = h.shape
    return pl.pallas_call(
        functools.partial(_node_body, apply_relu),
        out_shape=jax.ShapeDtypeStruct((n_pad, width), _F32),
        grid=(n_pad // tile,),
        in_specs=[pl.BlockSpec((2, 2, tile, _NBANK * width),
                               lambda i: (0, 0, i, 0)),
                  _tile_spec((tile, width)),
                  _const_spec(root.shape),
                  _const_spec(bias2d.shape)],
        out_specs=_tile_spec((tile, width)),
        compiler_params=pltpu.CompilerParams(
            dimension_semantics=("parallel",),
            vmem_limit_bytes=_VMEM_LIMIT,
        ),
    )(acc, h, root, bias2d)


@jax.jit
def _forward(fc1_w, fc1_b, k1_w, k1_b, k2_w, k2_b, k3_w, k3_b, root, bias,
             fc2_w, fc2_b, x, ea, src, tgt, scale):
    depth = 3
    edge_tile = 2048
    node_tile = 2048
    k_pad = ea.shape[1]
    ker_in = k1_w.shape[0]

    k1w = jnp.pad(k1_w, ((0, k_pad - ker_in), (0, 0)))
    k1b = k1_b.reshape(1, -1)
    k2b = k2_b.reshape(1, -1)
    k3b = k3_b.reshape(1, -1)
    bias2d = bias.reshape(1, -1)

    relu2 = _edge_mlp(ea, k1w, k1b, k2_w, k2b, tile=4096)

    width = root.shape[0]
    j = jnp.arange(width * width)
    rep = (jnp.arange(width)[:, None] == (j // width)[None, :]).astype(_F32)

    h = x * fc1_w[0][None, :] + fc1_b[None, :]

    src3d = src.reshape(-1, 1, edge_tile)
    tgt3d = tgt.reshape(-1, 1, edge_tile)

    for d in range(depth):
        acc = _messages_aggr(relu2, src3d, tgt3d, scale, h, k3_w, k3b, rep,
                             tile=edge_tile)
        h = _node_update(acc, h, root, bias2d,
                         tile=node_tile, apply_relu=(d != depth - 1))

    return h @ fc2_w + fc2_b[None, :]


def kernel(fc1_w, fc1_b, k1_w, k1_b, k2_w, k2_b, k3_w, k3_b, root, bias,
           fc2_w, fc2_b, x, ea, src, tgt, scale):
    return _forward(fc1_w, fc1_b, k1_w, k1_b, k2_w, k2_b, k3_w, k3_b, root,
                    bias, fc2_w, fc2_b, x, ea, src, tgt, scale)
